# baseline (speedup 1.0000x reference)
import functools

import jax
import jax.numpy as jnp
from jax import lax
from jax.experimental import pallas as pl
from jax.experimental.pallas import tpu as pltpu
from jax.experimental.pallas import tpu_sc as plsc

C = 0.01
SQRT_C = 0.1
EPS = 1e-06
N = 10000
E = 160000
F = 128
NR = 8
W = 160

NC = 2
NS = 16
NWORK = NC * NS
CH = 32
NBUF = 5
LK = 4
EW = ((E + NWORK * CH * NBUF - 1) // (NWORK * CH * NBUF)) * CH * NBUF
EP = EW * NWORK
NCHUNK = EW // CH
NP = 10016
ZR = 32
NPW = NP // NS

BN1 = 1000
BN3 = 1000


def _log_map_zero(x):
    nrm = jnp.maximum(jnp.sqrt(jnp.sum(x * x, axis=-1, keepdims=True)), 1e-10)
    t = jnp.clip(SQRT_C * nrm, -1.0 + 1e-07, 1.0 - 1e-07)
    artanh = 0.5 * jnp.log((1.0 + t) / (1.0 - t))
    return artanh * x / (SQRT_C * nrm)


def _exp_map_zero(v):
    nrm = jnp.maximum(jnp.sqrt(jnp.sum(v * v, axis=-1, keepdims=True)), 1e-10)
    return jnp.tanh(SQRT_C * nrm) * v / (SQRT_C * nrm)


def _dot(a, b):
    return lax.dot_general(a, b, (((1,), (0,)), ((), ())),
                           precision=lax.Precision.HIGHEST,
                           preferred_element_type=jnp.float32)


def _stage1_body(h_ref, rw_ref, re_ref, lw_ref, table_ref, loop_ref):
    h = h_ref[...]
    h_tan = _log_map_zero(h)
    for r in range(NR):
        y = _dot(h_tan, rw_ref[r]) + re_ref[r][None, :]
        yn = jnp.maximum(jnp.sqrt(jnp.sum(y * y, -1, keepdims=True)), 1e-10)
        msg = jnp.tanh(SQRT_C * yn) * y / (SQRT_C * yn)
        lam = 2.0 / (1.0 - C * jnp.sum(msg * msg, -1, keepdims=True) + EPS)
        row = jnp.concatenate(
            [lam * msg, lam, jnp.ones_like(lam),
             jnp.zeros((h.shape[0], W - F - 2), jnp.float32)], axis=-1)
        table_ref[r] = row.astype(jnp.bfloat16)
    loop_ref[...] = _exp_map_zero(_dot(h_tan, lw_ref[...]))


def _stage1(h_hyper, rel_weight, rel_emb, loop_weight):
    return pl.pallas_call(
        _stage1_body,
        grid=(N // BN1,),
        in_specs=[
            pl.BlockSpec((BN1, F), lambda i: (i, 0)),
            pl.BlockSpec((NR, F, F), lambda i: (0, 0, 0)),
            pl.BlockSpec((NR, F), lambda i: (0, 0)),
            pl.BlockSpec((F, F), lambda i: (0, 0)),
        ],
        out_specs=[
            pl.BlockSpec((NR, BN1, W), lambda i: (0, i, 0)),
            pl.BlockSpec((BN1, F), lambda i: (i, 0)),
        ],
        out_shape=[
            jax.ShapeDtypeStruct((NR, N, W), jnp.bfloat16),
            jax.ShapeDtypeStruct((N, F), jnp.float32),
        ],
    )(h_hyper, rel_weight, rel_emb, loop_weight)


def _sc_body(gidx_hbm, dst_hbm, table_hbm, out_hbm,
             gidx_v, didx_v, rows_v, zrow_v, accum,
             psem, zsem, gsem, ssem):
    cid = lax.axis_index("c")
    sid = lax.axis_index("s")
    ibase = (cid * NS + sid) * NCHUNK

    pltpu.async_copy(gidx_hbm.at[pl.ds(ibase, NCHUNK)], gidx_v, psem)
    pltpu.async_copy(dst_hbm.at[pl.ds(ibase, NCHUNK)], didx_v, psem)

    def _zrow(r, _):
        for j in range(W // 32):
            zrow_v[r, pl.ds(j * 32, 32)] = jnp.zeros((32,), jnp.bfloat16)
        return 0
    lax.fori_loop(0, ZR, _zrow, 0)

    zbase = sid * NPW
    NZF = NPW // ZR
    REM = NPW % ZR
    for k in range(NZF):
        pltpu.async_copy(zrow_v, accum.at[pl.ds(zbase + k * ZR, ZR)], zsem)
    if REM:
        pltpu.async_copy(zrow_v.at[pl.ds(0, REM)],
                         accum.at[pl.ds(zbase + NZF * ZR, REM)], zsem)
    for k in range(NZF):
        pltpu.make_async_copy(zrow_v, accum.at[pl.ds(zbase + k * ZR, ZR)],
                              zsem).wait()
    if REM:
        pltpu.make_async_copy(zrow_v.at[pl.ds(0, REM)],
                              accum.at[pl.ds(zbase + NZF * ZR, REM)],
                              zsem).wait()

    pltpu.make_async_copy(gidx_hbm.at[pl.ds(ibase, NCHUNK)],
                          gidx_v, psem).wait()
    pltpu.make_async_copy(dst_hbm.at[pl.ds(ibase, NCHUNK)],
                          didx_v, psem).wait()

    for b in range(LK):
        pltpu.async_copy(table_hbm.at[gidx_v.at[b]], rows_v.at[b], gsem.at[b])

    plsc.subcore_barrier()

    def _edge_pass(nch):
        def _grp(g, _):
            for i in range(NBUF):
                ch = g * NBUF + i
                pltpu.make_async_copy(table_hbm.at[gidx_v.at[ch]],
                                      rows_v.at[i], gsem.at[i]).wait()
                pltpu.async_copy(rows_v.at[i], accum.at[didx_v.at[ch]],
                                 ssem.at[i], add=True)
                bn = (i + LK) % NBUF

                @pl.when(ch + LK < nch)
                def _():
                    @pl.when(ch + LK - NBUF >= 0)
                    def _():
                        pltpu.make_async_copy(
                            rows_v.at[bn], accum.at[didx_v.at[ch + LK - NBUF]],
                            ssem.at[bn]).wait()
                    pltpu.async_copy(table_hbm.at[gidx_v.at[ch + LK]],
                                     rows_v.at[bn], gsem.at[bn])
            return 0
        lax.fori_loop(0, nch // NBUF, _grp, 0)

        for ch in range(nch - NBUF, nch):
            b = ch % NBUF
            pltpu.make_async_copy(rows_v.at[b], accum.at[didx_v.at[ch]],
                                  ssem.at[b]).wait()

    _edge_pass(NCHUNK)

    plsc.subcore_barrier()

    pltpu.sync_copy(accum.at[pl.ds(zbase, NPW)],
                    out_hbm.at[cid, pl.ds(zbase, NPW)])


def _stage2(gidx_p, dst_p, table):
    mesh = plsc.VectorSubcoreMesh(core_axis_name="c", subcore_axis_name="s",
                                  num_cores=NC, num_subcores=NS)
    k = functools.partial(
        pl.kernel,
        out_type=jax.ShapeDtypeStruct((NC, NP, W), jnp.bfloat16),
        mesh=mesh,
        compiler_params=pltpu.CompilerParams(use_tc_tiling_on_sc=False),
        scratch_types=[
            pltpu.VMEM((NCHUNK, CH), jnp.int32),
            pltpu.VMEM((NCHUNK, CH), jnp.int32),
            pltpu.VMEM((NBUF, CH, W), jnp.bfloat16),
            pltpu.VMEM((ZR, W), jnp.bfloat16),
            pltpu.VMEM_SHARED((NP, W), jnp.bfloat16),
            pltpu.SemaphoreType.DMA,
            pltpu.SemaphoreType.DMA,
            pltpu.SemaphoreType.DMA((NBUF,)),
            pltpu.SemaphoreType.DMA((NBUF,)),
        ],
    )(_sc_body)
    return k(gidx_p.reshape(EP // CH, CH), dst_p.reshape(EP // CH, CH), table)


def _stage3_body(p_ref, norm_ref, loop_ref, out_ref):
    acc = p_ref[0].astype(jnp.float32) + p_ref[1].astype(jnp.float32)
    s1 = acc[:, :F]
    s0 = acc[:, F:F + 1]
    cnt = acc[:, F + 1:F + 2]
    nrm = norm_ref[...]
    w = nrm / (cnt * nrm + 1e-06)
    ratio = (w * s1) / (w * s0 + EPS)
    rn = jnp.maximum(jnp.sqrt(jnp.sum(ratio * ratio, -1, keepdims=True)), 1e-10)
    max_norm = (1.0 - 1e-05) / SQRT_C
    h_agg = jnp.where(rn > max_norm, ratio * max_norm / rn, ratio)
    lp = loop_ref[...]
    x2 = jnp.sum(h_agg * h_agg, -1, keepdims=True)
    y2 = jnp.sum(lp * lp, -1, keepdims=True)
    xy = jnp.sum(h_agg * lp, -1, keepdims=True)
    num = (1.0 + 2.0 * C * xy + C * y2) * h_agg + (1.0 - C * x2) * lp
    den = 1.0 + 2.0 * C * xy + C * C * x2 * y2
    out_ref[...] = num / (den + 1e-15)


def _stage3(partials, norm, loop_hyp):
    return pl.pallas_call(
        _stage3_body,
        grid=(N // BN3,),
        in_specs=[
            pl.BlockSpec((NC, BN3, W), lambda i: (0, i, 0)),
            pl.BlockSpec((BN3, 1), lambda i: (i, 0)),
            pl.BlockSpec((BN3, F), lambda i: (i, 0)),
        ],
        out_specs=pl.BlockSpec((BN3, F), lambda i: (i, 0)),
        out_shape=jax.ShapeDtypeStruct((N, F), jnp.float32),
    )(partials, norm, loop_hyp)


def kernel(h_hyper, edge_index, edge_type, norm, rel_emb, rel_weight, loop_weight):
    table, loop_hyp = _stage1(h_hyper, rel_weight, rel_emb, loop_weight)
    table = table.reshape(NR * N, W)

    src = edge_index[0]
    dst = edge_index[1]
    pad = EP - E
    gidx_p = jnp.concatenate([edge_type * N + src, jnp.zeros((pad,), jnp.int32)])
    dst_p = jnp.concatenate([dst, jnp.full((pad,), N, jnp.int32)])

    partials = _stage2(gidx_p, dst_p, table)
    return _stage3(partials, norm, loop_hyp)


# speedup vs baseline: 1.1221x; 1.1221x over previous
import functools

import jax
import jax.numpy as jnp
from jax import lax
from jax.experimental import pallas as pl
from jax.experimental.pallas import tpu as pltpu
from jax.experimental.pallas import tpu_sc as plsc

C = 0.01
SQRT_C = 0.1
EPS = 1e-06
N = 10000
E = 160000
F = 128
NR = 8
W = 144

NC = 2
NS = 16
NWORK = NC * NS
CH = 32
NBUF = 5
LK = 4
EW = ((E + NWORK * CH * NBUF - 1) // (NWORK * CH * NBUF)) * CH * NBUF
EP = EW * NWORK
NCHUNK = EW // CH
NP = 10016
ZR = 32
NPW = NP // NS

BN1 = 2000
BN3 = 2000


def _log_map_zero(x):
    nrm = jnp.maximum(jnp.sqrt(jnp.sum(x * x, axis=-1, keepdims=True)), 1e-10)
    t = jnp.clip(SQRT_C * nrm, -1.0 + 1e-07, 1.0 - 1e-07)
    artanh = 0.5 * jnp.log((1.0 + t) / (1.0 - t))
    return artanh * x / (SQRT_C * nrm)


def _exp_map_zero(v):
    nrm = jnp.maximum(jnp.sqrt(jnp.sum(v * v, axis=-1, keepdims=True)), 1e-10)
    return jnp.tanh(SQRT_C * nrm) * v / (SQRT_C * nrm)


def _dot(a, b):
    return lax.dot_general(a, b, (((1,), (0,)), ((), ())),
                           precision=lax.Precision.HIGHEST,
                           preferred_element_type=jnp.float32)


def _stage1_body(h_ref, rw_ref, re_ref, lw_ref, table_ref, loop_ref):
    h = h_ref[...]
    h_tan = _log_map_zero(h)
    for r in range(NR):
        y = _dot(h_tan, rw_ref[r]) + re_ref[r][None, :]
        yn = jnp.maximum(jnp.sqrt(jnp.sum(y * y, -1, keepdims=True)), 1e-10)
        msg = jnp.tanh(SQRT_C * yn) * y / (SQRT_C * yn)
        lam = 2.0 / (1.0 - C * jnp.sum(msg * msg, -1, keepdims=True) + EPS)
        row = jnp.concatenate(
            [lam * msg, lam, jnp.ones_like(lam),
             jnp.zeros((h.shape[0], W - F - 2), jnp.float32)], axis=-1)
        table_ref[r] = row
    loop_ref[...] = _exp_map_zero(_dot(h_tan, lw_ref[...]))


def _stage1(h_hyper, rel_weight, rel_emb, loop_weight):
    return pl.pallas_call(
        _stage1_body,
        grid=(N // BN1,),
        in_specs=[
            pl.BlockSpec((BN1, F), lambda i: (i, 0)),
            pl.BlockSpec((NR, F, F), lambda i: (0, 0, 0)),
            pl.BlockSpec((NR, F), lambda i: (0, 0)),
            pl.BlockSpec((F, F), lambda i: (0, 0)),
        ],
        out_specs=[
            pl.BlockSpec((NR, BN1, W), lambda i: (0, i, 0)),
            pl.BlockSpec((BN1, F), lambda i: (i, 0)),
        ],
        out_shape=[
            jax.ShapeDtypeStruct((NR, N, W), jnp.float32),
            jax.ShapeDtypeStruct((N, F), jnp.float32),
        ],
    )(h_hyper, rel_weight, rel_emb, loop_weight)


def _sc_body(gidx_hbm, dst_hbm, table_hbm, out_hbm,
             gidx_v, didx_v, rows_v, zrow_v, accum,
             psem, zsem, gsem, ssem):
    cid = lax.axis_index("c")
    sid = lax.axis_index("s")
    ibase = (cid * NS + sid) * NCHUNK

    pltpu.async_copy(gidx_hbm.at[pl.ds(ibase, NCHUNK)], gidx_v, psem)
    pltpu.async_copy(dst_hbm.at[pl.ds(ibase, NCHUNK)], didx_v, psem)

    def _zrow(r, _):
        for j in range(W // 16):
            zrow_v[r, pl.ds(j * 16, 16)] = jnp.zeros((16,), jnp.float32)
        return 0
    lax.fori_loop(0, ZR, _zrow, 0)

    zbase = sid * NPW
    NZF = NPW // ZR
    REM = NPW % ZR
    for k in range(NZF):
        pltpu.async_copy(zrow_v, accum.at[pl.ds(zbase + k * ZR, ZR)], zsem)
    if REM:
        pltpu.async_copy(zrow_v.at[pl.ds(0, REM)],
                         accum.at[pl.ds(zbase + NZF * ZR, REM)], zsem)
    for k in range(NZF):
        pltpu.make_async_copy(zrow_v, accum.at[pl.ds(zbase + k * ZR, ZR)],
                              zsem).wait()
    if REM:
        pltpu.make_async_copy(zrow_v.at[pl.ds(0, REM)],
                              accum.at[pl.ds(zbase + NZF * ZR, REM)],
                              zsem).wait()

    pltpu.make_async_copy(gidx_hbm.at[pl.ds(ibase, NCHUNK)],
                          gidx_v, psem).wait()
    pltpu.make_async_copy(dst_hbm.at[pl.ds(ibase, NCHUNK)],
                          didx_v, psem).wait()

    for b in range(LK):
        pltpu.async_copy(table_hbm.at[gidx_v.at[b]], rows_v.at[b], gsem.at[b])

    plsc.subcore_barrier()

    def _edge_pass(nch):
        def _grp(g, _):
            for i in range(NBUF):
                ch = g * NBUF + i
                pltpu.make_async_copy(table_hbm.at[gidx_v.at[ch]],
                                      rows_v.at[i], gsem.at[i]).wait()
                pltpu.async_copy(rows_v.at[i], accum.at[didx_v.at[ch]],
                                 ssem.at[i], add=True)
                bn = (i + LK) % NBUF

                @pl.when(ch + LK < nch)
                def _():
                    @pl.when(ch + LK - NBUF >= 0)
                    def _():
                        pltpu.make_async_copy(
                            rows_v.at[bn], accum.at[didx_v.at[ch + LK - NBUF]],
                            ssem.at[bn]).wait()
                    pltpu.async_copy(table_hbm.at[gidx_v.at[ch + LK]],
                                     rows_v.at[bn], gsem.at[bn])
            return 0
        lax.fori_loop(0, nch // NBUF, _grp, 0)

        for ch in range(nch - NBUF, nch):
            b = ch % NBUF
            pltpu.make_async_copy(rows_v.at[b], accum.at[didx_v.at[ch]],
                                  ssem.at[b]).wait()

    _edge_pass(NCHUNK)

    plsc.subcore_barrier()

    pltpu.sync_copy(accum.at[pl.ds(zbase, NPW)],
                    out_hbm.at[cid, pl.ds(zbase, NPW)])


def _stage2(gidx_p, dst_p, table):
    mesh = plsc.VectorSubcoreMesh(core_axis_name="c", subcore_axis_name="s",
                                  num_cores=NC, num_subcores=NS)
    k = functools.partial(
        pl.kernel,
        out_type=jax.ShapeDtypeStruct((NC, NP, W), jnp.float32),
        mesh=mesh,
        compiler_params=pltpu.CompilerParams(use_tc_tiling_on_sc=False),
        scratch_types=[
            pltpu.VMEM((NCHUNK, CH), jnp.int32),
            pltpu.VMEM((NCHUNK, CH), jnp.int32),
            pltpu.VMEM((NBUF, CH, W), jnp.float32),
            pltpu.VMEM((ZR, W), jnp.float32),
            pltpu.VMEM_SHARED((NP, W), jnp.float32),
            pltpu.SemaphoreType.DMA,
            pltpu.SemaphoreType.DMA,
            pltpu.SemaphoreType.DMA((NBUF,)),
            pltpu.SemaphoreType.DMA((NBUF,)),
        ],
    )(_sc_body)
    return k(gidx_p.reshape(EP // CH, CH), dst_p.reshape(EP // CH, CH), table)


def _stage3_body(p_ref, norm_ref, loop_ref, out_ref):
    acc = p_ref[0] + p_ref[1]
    s1 = acc[:, :F]
    s0 = acc[:, F:F + 1]
    cnt = acc[:, F + 1:F + 2]
    nrm = norm_ref[...]
    w = nrm / (cnt * nrm + 1e-06)
    ratio = (w * s1) / (w * s0 + EPS)
    rn = jnp.maximum(jnp.sqrt(jnp.sum(ratio * ratio, -1, keepdims=True)), 1e-10)
    max_norm = (1.0 - 1e-05) / SQRT_C
    h_agg = jnp.where(rn > max_norm, ratio * max_norm / rn, ratio)
    lp = loop_ref[...]
    x2 = jnp.sum(h_agg * h_agg, -1, keepdims=True)
    y2 = jnp.sum(lp * lp, -1, keepdims=True)
    xy = jnp.sum(h_agg * lp, -1, keepdims=True)
    num = (1.0 + 2.0 * C * xy + C * y2) * h_agg + (1.0 - C * x2) * lp
    den = 1.0 + 2.0 * C * xy + C * C * x2 * y2
    out_ref[...] = num / (den + 1e-15)


def _stage3(partials, norm, loop_hyp):
    return pl.pallas_call(
        _stage3_body,
        grid=(N // BN3,),
        in_specs=[
            pl.BlockSpec((NC, BN3, W), lambda i: (0, i, 0)),
            pl.BlockSpec((BN3, 1), lambda i: (i, 0)),
            pl.BlockSpec((BN3, F), lambda i: (i, 0)),
        ],
        out_specs=pl.BlockSpec((BN3, F), lambda i: (i, 0)),
        out_shape=jax.ShapeDtypeStruct((N, F), jnp.float32),
    )(partials, norm, loop_hyp)


def kernel(h_hyper, edge_index, edge_type, norm, rel_emb, rel_weight, loop_weight):
    table, loop_hyp = _stage1(h_hyper, rel_weight, rel_emb, loop_weight)
    table = table.reshape(NR * N, W)

    src = edge_index[0]
    dst = edge_index[1]
    pad = EP - E
    gidx_p = jnp.concatenate([edge_type * N + src, jnp.zeros((pad,), jnp.int32)])
    dst_p = jnp.concatenate([dst, jnp.full((pad,), N, jnp.int32)])

    partials = _stage2(gidx_p, dst_p, table)
    return _stage3(partials, norm, loop_hyp)
